# baseline (speedup 1.0000x reference)
import jax
import jax.numpy as jnp
import numpy as np
from jax import lax
from jax.experimental import pallas as pl
from jax.experimental.pallas import tpu as pltpu
from jax.experimental.pallas import tpu_sc as plsc

_N = 10000
_E = 320000
_D = 128
_H = 16
_C = 40

_NC, _NS = 2, 16
_NW = _NC * _NS
_CH = 128
_NCHUNK = 80
_NMAIN = 78
_NP = 10112
_NV = _NP // 8
_RPT = _NP // _NS
_GRID = 2
_VB = _NV // _GRID

_NTAIL = _NW * _NCHUNK - 2500
_TRASH2 = np.stack([
    np.zeros((_NTAIL, _CH), np.int32),
    (_N + np.arange(_NTAIL * _CH, dtype=np.int64) % (_NP - _N)
     ).reshape(_NTAIL, _CH).astype(np.int32),
])


def _mm_body(x_ref, wl_ref, wr_ref, y_ref, xr_ref):
  ycols, xcols = [], []
  for k in range(8):
    xk = x_ref[:, k, :]
    ycols.append(jnp.dot(xk, wl_ref[...],
                         preferred_element_type=jnp.float32))
    xcols.append(jnp.dot(xk, wr_ref[...],
                         preferred_element_type=jnp.float32))
  y_ref[...] = jnp.concatenate(ycols, axis=1)
  xr_ref[...] = jnp.concatenate(xcols, axis=1)


_mm1 = pl.pallas_call(
    _mm_body,
    grid=(_GRID,),
    in_specs=[
        pl.BlockSpec((_VB, 8, _D), lambda i: (i, 0, 0)),
        pl.BlockSpec((_D, _H), lambda i: (0, 0)),
        pl.BlockSpec((_D, _H), lambda i: (0, 0)),
    ],
    out_specs=[
        pl.BlockSpec((_VB, 128), lambda i: (i, 0)),
        pl.BlockSpec((_VB, 128), lambda i: (i, 0)),
    ],
    out_shape=[
        jax.ShapeDtypeStruct((_NV, 128), jnp.float32),
        jax.ShapeDtypeStruct((_NV, 128), jnp.float32),
    ],
)


_DEPTH = 4


def _load_indices(edge3, pad64, row, buf, wid):
  pltpu.sync_copy(edge3.at[row, pl.ds(wid * _NMAIN, _NMAIN)],
                  buf.at[pl.ds(0, _NMAIN)])
  pltpu.sync_copy(pad64.at[row, pl.ds(2 * wid, 2)],
                  buf.at[pl.ds(_NMAIN, 2)])


def _edge_loop(table, src_v, rows, gsems, scatter):
  for p in range(_DEPTH):
    pltpu.async_copy(table.at[src_v.at[p]], rows[p], gsems[p])

  def group(i, carry):
    j = _DEPTH * i
    descs = []
    for p in range(_DEPTH):
      pltpu.make_async_copy(table.at[src_v.at[j + p]], rows[p],
                            gsems[p]).wait()
      descs.append(scatter(rows[p], j + p, p))
    for p in range(_DEPTH):
      descs[p].wait()
      pltpu.async_copy(table.at[src_v.at[j + _DEPTH + p]], rows[p], gsems[p])
    return carry

  lax.fori_loop(0, _NCHUNK // _DEPTH - 1, group, 0)
  j = _NCHUNK - _DEPTH
  for p in range(_DEPTH):
    pltpu.make_async_copy(table.at[src_v.at[j + p]], rows[p], gsems[p]).wait()
    scatter(rows[p], j + p, p).wait()


def _sc_l1_body(table, edge3, pad64, zeros, ones, agg_o, deg_o,
                src_v, dst_v, r0_v, r1_v, r2_v, r3_v, r4_v, r5_v, r6_v,
                r7_v, ones_v, acc, dega,
                g0, g1, g2, g3, g4, g5, g6, g7,
                s0, s1, s2, s3, s4, s5, s6, s7, dsem):
  cid = lax.axis_index("c")
  sid = lax.axis_index("s")
  wid = sid * _NC + cid
  n0 = sid * _RPT
  rows = [r0_v, r1_v, r2_v, r3_v, r4_v, r5_v, r6_v, r7_v]
  gsems = [g0, g1, g2, g3, g4, g5, g6, g7]
  ssems = [s0, s1, s2, s3, s4, s5, s6, s7]
  pltpu.sync_copy(zeros.at[pl.ds(n0, _RPT)], acc.at[pl.ds(n0, _RPT)])
  pltpu.sync_copy(zeros.at[pl.ds(n0, _RPT)], dega.at[pl.ds(n0, _RPT)])
  pltpu.sync_copy(ones, ones_v)
  _load_indices(edge3, pad64, 0, src_v, wid)
  _load_indices(edge3, pad64, 1, dst_v, wid)
  plsc.subcore_barrier()

  def scatter(rows_buf, j, p):
    d = pltpu.async_copy(rows_buf, acc.at[dst_v.at[j]], ssems[p], add=True)
    pltpu.async_copy(ones_v, dega.at[dst_v.at[j]], dsem, add=True)
    return d

  _edge_loop(table, src_v, rows, gsems, scatter)

  def drain(i, carry):
    pltpu.make_async_copy(ones, ones_v, dsem).wait()
    return carry

  lax.fori_loop(0, _NCHUNK, drain, 0)
  plsc.subcore_barrier()
  pltpu.sync_copy(acc.at[pl.ds(n0, _RPT)], agg_o.at[cid, pl.ds(n0, _RPT)])
  pltpu.sync_copy(dega.at[pl.ds(n0, _RPT)], deg_o.at[cid, pl.ds(n0, _RPT)])


def _sc_l2_body(table, edge3, pad64, zeros, agg_o,
                src_v, dst_v, r0_v, r1_v, r2_v, r3_v, r4_v, r5_v, r6_v,
                r7_v, acc,
                g0, g1, g2, g3, g4, g5, g6, g7,
                s0, s1, s2, s3, s4, s5, s6, s7):
  cid = lax.axis_index("c")
  sid = lax.axis_index("s")
  wid = sid * _NC + cid
  n0 = sid * _RPT
  rows = [r0_v, r1_v, r2_v, r3_v, r4_v, r5_v, r6_v, r7_v]
  gsems = [g0, g1, g2, g3, g4, g5, g6, g7]
  ssems = [s0, s1, s2, s3, s4, s5, s6, s7]
  pltpu.sync_copy(zeros.at[pl.ds(n0, _RPT)], acc.at[pl.ds(n0, _RPT)])
  _load_indices(edge3, pad64, 0, src_v, wid)
  _load_indices(edge3, pad64, 1, dst_v, wid)
  plsc.subcore_barrier()

  def scatter(rows_buf, j, p):
    return pltpu.async_copy(rows_buf, acc.at[dst_v.at[j]], ssems[p],
                            add=True)

  _edge_loop(table, src_v, rows, gsems, scatter)
  plsc.subcore_barrier()
  pltpu.sync_copy(acc.at[pl.ds(n0, _RPT)], agg_o.at[cid, pl.ds(n0, _RPT)])


_sc_mesh = plsc.VectorSubcoreMesh(core_axis_name="c", subcore_axis_name="s")
_sc_params = pltpu.CompilerParams(use_tc_tiling_on_sc=False)

_sc_l1 = pl.kernel(
    _sc_l1_body,
    compiler_params=_sc_params,
    out_type=[
        jax.ShapeDtypeStruct((_NC, _NP, _H), jnp.float32),
        jax.ShapeDtypeStruct((_NC, _NP, _H), jnp.float32),
    ],
    mesh=_sc_mesh,
    scratch_types=[
        pltpu.VMEM((_NCHUNK, _CH), jnp.int32),
        pltpu.VMEM((_NCHUNK, _CH), jnp.int32),
    ] + [pltpu.VMEM((_CH, _H), jnp.float32)] * 9 + [
        pltpu.VMEM_SHARED((_NP, _H), jnp.float32),
        pltpu.VMEM_SHARED((_NP, _H), jnp.float32),
    ] + [pltpu.SemaphoreType.DMA] * 17,
)

_sc_l2 = pl.kernel(
    _sc_l2_body,
    compiler_params=_sc_params,
    out_type=jax.ShapeDtypeStruct((_NC, _NP, _H), jnp.float32),
    mesh=_sc_mesh,
    scratch_types=[
        pltpu.VMEM((_NCHUNK, _CH), jnp.int32),
        pltpu.VMEM((_NCHUNK, _CH), jnp.int32),
    ] + [pltpu.VMEM((_CH, _H), jnp.float32)] * 8 + [
        pltpu.VMEM_SHARED((_NP, _H), jnp.float32),
    ] + [pltpu.SemaphoreType.DMA] * 16,
)


def _tc2_body(aggp, degp, xr, b1, h_o):
  agg = aggp[0] + aggp[1]
  deg = jnp.maximum(degp[0] + degp[1], 1.0)
  h_o[...] = jnp.maximum(agg / deg + b1[...] + xr[...], 0.0)


_tc2 = pl.pallas_call(
    _tc2_body,
    grid=(_GRID,),
    in_specs=[
        pl.BlockSpec((_NC, _VB, 128), lambda i: (0, i, 0)),
        pl.BlockSpec((_NC, _VB, 128), lambda i: (0, i, 0)),
        pl.BlockSpec((_VB, 128), lambda i: (i, 0)),
        pl.BlockSpec((1, 128), lambda i: (0, 0)),
    ],
    out_specs=pl.BlockSpec((_VB, 128), lambda i: (i, 0)),
    out_shape=jax.ShapeDtypeStruct((_NV, 128), jnp.float32),
)


def _tc3_body(aggp, degp, hv, bdwl2, bdwr2, b2, o_ref):
  mean2 = (aggp[0] + aggp[1]) / jnp.maximum(degp[0] + degp[1], 1.0)
  z = (jnp.dot(mean2, bdwl2[...], preferred_element_type=jnp.float32)
       + jnp.dot(hv[...], bdwr2[...], preferred_element_type=jnp.float32)
       + b2[...])
  outs = []
  for g in range(8):
    zg = z[:, _C * g:_C * (g + 1)]
    m = jnp.max(zg, axis=1, keepdims=True)
    s = jnp.sum(jnp.exp(zg - m), axis=1, keepdims=True)
    outs.append(zg - m - jnp.log(s))
  o_ref[...] = jnp.concatenate(outs, axis=1)


_tc3 = pl.pallas_call(
    _tc3_body,
    grid=(_GRID,),
    in_specs=[
        pl.BlockSpec((_NC, _VB, 128), lambda i: (0, i, 0)),
        pl.BlockSpec((_NC, _VB, 128), lambda i: (0, i, 0)),
        pl.BlockSpec((_VB, 128), lambda i: (i, 0)),
        pl.BlockSpec((128, 8 * _C), lambda i: (0, 0)),
        pl.BlockSpec((128, 8 * _C), lambda i: (0, 0)),
        pl.BlockSpec((1, 8 * _C), lambda i: (0, 0)),
    ],
    out_specs=pl.BlockSpec((_VB, 8 * _C), lambda i: (i, 0)),
    out_shape=jax.ShapeDtypeStruct((_NV, 8 * _C), jnp.float32),
)


def kernel(x, edge_index, W_l1, b_l1, W_r1, W_l2, b_l2, W_r2):
  x3 = jnp.pad(x, ((0, _NP - _N), (0, 0))).reshape(_NV, 8, _D)
  edge3 = edge_index.reshape(2, _E // _CH, _CH)
  tail = edge_index[:, _NMAIN * _NW * _CH:].reshape(2, -1, _CH)
  pad64 = jnp.concatenate([tail, jnp.asarray(_TRASH2)], axis=1)
  zeros = jnp.zeros((_NP, _H), jnp.float32)
  ones = jnp.ones((_CH, _H), jnp.float32)
  eye8 = jnp.eye(8, dtype=jnp.float32)
  bdwl2 = jnp.kron(eye8, W_l2)
  bdwr2 = jnp.kron(eye8, W_r2)
  b1t = jnp.tile(b_l1, 8).reshape(1, 128)
  b2t = jnp.tile(b_l2, 8).reshape(1, 8 * _C)

  y1v, xrv = _mm1(x3, W_l1, W_r1)
  aggp, degp = _sc_l1(y1v.reshape(_NP, _H), edge3, pad64, zeros, ones)
  aggv = aggp.reshape(_NC, _NV, 128)
  degv = degp.reshape(_NC, _NV, 128)
  hv = _tc2(aggv, degv, xrv, b1t)
  agg2p = _sc_l2(hv.reshape(_NP, _H), edge3, pad64, zeros)
  outv = _tc3(agg2p.reshape(_NC, _NV, 128), degv, hv, bdwl2, bdwr2, b2t)
  return outv.reshape(_NP, _C)[:_N]


# speedup vs baseline: 1.5213x; 1.5213x over previous
import jax
import jax.numpy as jnp
import numpy as np
from jax import lax
from jax.experimental import pallas as pl
from jax.experimental.pallas import tpu as pltpu
from jax.experimental.pallas import tpu_sc as plsc

_N = 10000
_E = 320000
_D = 128
_H = 16
_C = 40

_NC, _NS = 2, 16
_NW = _NC * _NS
_CH = 128
_NCHUNK = 80
_NMAIN = 78
_NP = 10112
_NV = _NP // 8
_RPT = _NP // _NS
_GRID = 2
_VB = _NV // _GRID

_NTAIL = _NW * _NCHUNK - 2500
_TRASH = np.asarray(
    (_N + np.arange(_NTAIL * _CH) % (_NP - _N)).reshape(1, _NTAIL, _CH),
    np.int32)
_TRASH2 = np.broadcast_to(_TRASH, (2, _NTAIL, _CH))


def _mm_body(x_ref, wl_ref, wr_ref, y_ref, xr_ref):
  ycols, xcols = [], []
  for k in range(8):
    xk = x_ref[:, k, :]
    ycols.append(jnp.dot(xk, wl_ref[...],
                         preferred_element_type=jnp.float32))
    xcols.append(jnp.dot(xk, wr_ref[...],
                         preferred_element_type=jnp.float32))
  y_ref[...] = jnp.concatenate(ycols, axis=1)
  xr_ref[...] = jnp.concatenate(xcols, axis=1)


_mm1 = pl.pallas_call(
    _mm_body,
    grid=(_GRID,),
    in_specs=[
        pl.BlockSpec((_VB, 8, _D), lambda i: (i, 0, 0)),
        pl.BlockSpec((_D, _H), lambda i: (0, 0)),
        pl.BlockSpec((_D, _H), lambda i: (0, 0)),
    ],
    out_specs=[
        pl.BlockSpec((_VB, 128), lambda i: (i, 0)),
        pl.BlockSpec((_VB, 128), lambda i: (i, 0)),
    ],
    out_shape=[
        jax.ShapeDtypeStruct((_NV, 128), jnp.float32),
        jax.ShapeDtypeStruct((_NV, 128), jnp.float32),
    ],
)


_DEPTH = 4


def _load_indices(edge3, pad64, row, buf, wid):
  pltpu.sync_copy(edge3.at[row, pl.ds(wid * _NMAIN, _NMAIN)],
                  buf.at[pl.ds(0, _NMAIN)])
  pltpu.sync_copy(pad64.at[row, pl.ds(2 * wid, 2)],
                  buf.at[pl.ds(_NMAIN, 2)])


def _edge_loop(table, src_v, rows, gsems, scatter):
  for p in range(_DEPTH):
    pltpu.async_copy(table.at[src_v.at[p]], rows[p], gsems[p])

  def group(i, carry):
    j = _DEPTH * i
    descs = []
    for p in range(_DEPTH):
      pltpu.make_async_copy(table.at[src_v.at[j + p]], rows[p],
                            gsems[p]).wait()
      descs.append(scatter(rows[p], j + p, p))
    for p in range(_DEPTH):
      descs[p].wait()
      pltpu.async_copy(table.at[src_v.at[j + _DEPTH + p]], rows[p], gsems[p])
    return carry

  lax.fori_loop(0, _NCHUNK // _DEPTH - 1, group, 0)
  j = _NCHUNK - _DEPTH
  for p in range(_DEPTH):
    pltpu.make_async_copy(table.at[src_v.at[j + p]], rows[p], gsems[p]).wait()
    scatter(rows[p], j + p, p).wait()


def _sc_l1_body(table, edge3, pad64, zeros, ones, agg_o, deg_o,
                src_v, dst_v, r0_v, r1_v, r2_v, r3_v, r4_v, r5_v, r6_v,
                r7_v, ones_v, acc, dega,
                g0, g1, g2, g3, g4, g5, g6, g7,
                s0, s1, s2, s3, s4, s5, s6, s7, dsem):
  cid = lax.axis_index("c")
  sid = lax.axis_index("s")
  wid = sid * _NC + cid
  n0 = sid * _RPT
  rows = [r0_v, r1_v, r2_v, r3_v, r4_v, r5_v, r6_v, r7_v]
  gsems = [g0, g1, g2, g3, g4, g5, g6, g7]
  ssems = [s0, s1, s2, s3, s4, s5, s6, s7]
  pltpu.sync_copy(zeros.at[pl.ds(n0, _RPT)], acc.at[pl.ds(n0, _RPT)])
  pltpu.sync_copy(zeros.at[pl.ds(n0, _RPT)], dega.at[pl.ds(n0, _RPT)])
  pltpu.sync_copy(ones, ones_v)
  _load_indices(edge3, pad64, 0, src_v, wid)
  _load_indices(edge3, pad64, 1, dst_v, wid)
  plsc.subcore_barrier()

  def scatter(rows_buf, j, p):
    d = pltpu.async_copy(rows_buf, acc.at[dst_v.at[j]], ssems[p], add=True)
    pltpu.async_copy(ones_v, dega.at[dst_v.at[j]], dsem, add=True)
    return d

  _edge_loop(table, src_v, rows, gsems, scatter)

  def drain(i, carry):
    pltpu.make_async_copy(ones, ones_v, dsem).wait()
    return carry

  lax.fori_loop(0, _NCHUNK, drain, 0)
  plsc.subcore_barrier()
  pltpu.sync_copy(acc.at[pl.ds(n0, _RPT)], agg_o.at[cid, pl.ds(n0, _RPT)])
  pltpu.sync_copy(dega.at[pl.ds(n0, _RPT)], deg_o.at[cid, pl.ds(n0, _RPT)])


def _sc_l2_body(table, edge3, pad64, zeros, agg_o,
                src_v, dst_v, r0_v, r1_v, r2_v, r3_v, r4_v, r5_v, r6_v,
                r7_v, acc,
                g0, g1, g2, g3, g4, g5, g6, g7,
                s0, s1, s2, s3, s4, s5, s6, s7):
  cid = lax.axis_index("c")
  sid = lax.axis_index("s")
  wid = sid * _NC + cid
  n0 = sid * _RPT
  rows = [r0_v, r1_v, r2_v, r3_v, r4_v, r5_v, r6_v, r7_v]
  gsems = [g0, g1, g2, g3, g4, g5, g6, g7]
  ssems = [s0, s1, s2, s3, s4, s5, s6, s7]
  pltpu.sync_copy(zeros.at[pl.ds(n0, _RPT)], acc.at[pl.ds(n0, _RPT)])
  _load_indices(edge3, pad64, 0, src_v, wid)
  _load_indices(edge3, pad64, 1, dst_v, wid)
  plsc.subcore_barrier()

  def scatter(rows_buf, j, p):
    return pltpu.async_copy(rows_buf, acc.at[dst_v.at[j]], ssems[p],
                            add=True)

  _edge_loop(table, src_v, rows, gsems, scatter)
  plsc.subcore_barrier()
  pltpu.sync_copy(acc.at[pl.ds(n0, _RPT)], agg_o.at[cid, pl.ds(n0, _RPT)])


_sc_mesh = plsc.VectorSubcoreMesh(core_axis_name="c", subcore_axis_name="s")
_sc_params = pltpu.CompilerParams(use_tc_tiling_on_sc=False)

_sc_l1 = pl.kernel(
    _sc_l1_body,
    compiler_params=_sc_params,
    out_type=[
        jax.ShapeDtypeStruct((_NC, _NP, _H), jnp.float32),
        jax.ShapeDtypeStruct((_NC, _NP, _H), jnp.float32),
    ],
    mesh=_sc_mesh,
    scratch_types=[
        pltpu.VMEM((_NCHUNK, _CH), jnp.int32),
        pltpu.VMEM((_NCHUNK, _CH), jnp.int32),
    ] + [pltpu.VMEM((_CH, _H), jnp.float32)] * 9 + [
        pltpu.VMEM_SHARED((_NP, _H), jnp.float32),
        pltpu.VMEM_SHARED((_NP, _H), jnp.float32),
    ] + [pltpu.SemaphoreType.DMA] * 17,
)

_sc_l2 = pl.kernel(
    _sc_l2_body,
    compiler_params=_sc_params,
    out_type=jax.ShapeDtypeStruct((_NC, _NP, _H), jnp.float32),
    mesh=_sc_mesh,
    scratch_types=[
        pltpu.VMEM((_NCHUNK, _CH), jnp.int32),
        pltpu.VMEM((_NCHUNK, _CH), jnp.int32),
    ] + [pltpu.VMEM((_CH, _H), jnp.float32)] * 8 + [
        pltpu.VMEM_SHARED((_NP, _H), jnp.float32),
    ] + [pltpu.SemaphoreType.DMA] * 16,
)


def _tc2_body(aggp, degp, xr, b1, h_o):
  agg = aggp[0] + aggp[1]
  deg = jnp.maximum(degp[0] + degp[1], 1.0)
  h_o[...] = jnp.maximum(agg / deg + b1[...] + xr[...], 0.0)


_tc2 = pl.pallas_call(
    _tc2_body,
    grid=(_GRID,),
    in_specs=[
        pl.BlockSpec((_NC, _VB, 128), lambda i: (0, i, 0)),
        pl.BlockSpec((_NC, _VB, 128), lambda i: (0, i, 0)),
        pl.BlockSpec((_VB, 128), lambda i: (i, 0)),
        pl.BlockSpec((1, 128), lambda i: (0, 0)),
    ],
    out_specs=pl.BlockSpec((_VB, 128), lambda i: (i, 0)),
    out_shape=jax.ShapeDtypeStruct((_NV, 128), jnp.float32),
)


def _tc3_body(aggp, degp, hv, bdwl2, bdwr2, b2, o_ref):
  mean2 = (aggp[0] + aggp[1]) / jnp.maximum(degp[0] + degp[1], 1.0)
  z = (jnp.dot(mean2, bdwl2[...], preferred_element_type=jnp.float32)
       + jnp.dot(hv[...], bdwr2[...], preferred_element_type=jnp.float32)
       + b2[...])
  outs = []
  for g in range(8):
    zg = z[:, _C * g:_C * (g + 1)]
    m = jnp.max(zg, axis=1, keepdims=True)
    s = jnp.sum(jnp.exp(zg - m), axis=1, keepdims=True)
    outs.append(zg - m - jnp.log(s))
  o_ref[...] = jnp.concatenate(outs, axis=1)


_tc3 = pl.pallas_call(
    _tc3_body,
    grid=(_GRID,),
    in_specs=[
        pl.BlockSpec((_NC, _VB, 128), lambda i: (0, i, 0)),
        pl.BlockSpec((_NC, _VB, 128), lambda i: (0, i, 0)),
        pl.BlockSpec((_VB, 128), lambda i: (i, 0)),
        pl.BlockSpec((128, 8 * _C), lambda i: (0, 0)),
        pl.BlockSpec((128, 8 * _C), lambda i: (0, 0)),
        pl.BlockSpec((1, 8 * _C), lambda i: (0, 0)),
    ],
    out_specs=pl.BlockSpec((_VB, 8 * _C), lambda i: (i, 0)),
    out_shape=jax.ShapeDtypeStruct((_NV, 8 * _C), jnp.float32),
)


def kernel(x, edge_index, W_l1, b_l1, W_r1, W_l2, b_l2, W_r2):
  x3 = jnp.pad(x, ((0, _NP - _N), (0, 0))).reshape(_NV, 8, _D)
  edge3 = edge_index.reshape(2, _E // _CH, _CH)
  tail = edge_index[:, _NMAIN * _NW * _CH:].reshape(2, -1, _CH)
  pad64 = jnp.concatenate([tail, jnp.asarray(_TRASH2)], axis=1)
  zeros = jnp.zeros((_NP, _H), jnp.float32)
  ones = jnp.ones((_CH, _H), jnp.float32)
  eye8 = jnp.eye(8, dtype=jnp.float32)
  bdwl2 = jnp.kron(eye8, W_l2)
  bdwr2 = jnp.kron(eye8, W_r2)
  b1t = jnp.tile(b_l1, 8).reshape(1, 128)
  b2t = jnp.tile(b_l2, 8).reshape(1, 8 * _C)

  y1v, xrv = _mm1(x3, W_l1, W_r1)
  aggp, degp = _sc_l1(y1v.reshape(_NP, _H), edge3, pad64, zeros, ones)
  aggv = aggp.reshape(_NC, _NV, 128)
  degv = degp.reshape(_NC, _NV, 128)
  hv = _tc2(aggv, degv, xrv, b1t)
  agg2p = _sc_l2(hv.reshape(_NP, _H), edge3, pad64, zeros)
  outv = _tc3(agg2p.reshape(_NC, _NV, 128), degv, hv, bdwl2, bdwr2, b2t)
  return outv.reshape(_NP, _C)[:_N]


# speedup vs baseline: 1.6673x; 1.0960x over previous
import jax
import jax.numpy as jnp
import numpy as np
from jax import lax
from jax.experimental import pallas as pl
from jax.experimental.pallas import tpu as pltpu
from jax.experimental.pallas import tpu_sc as plsc

_N = 10000
_E = 320000
_D = 128
_H = 16
_C = 40

_NC, _NS = 2, 16
_NW = _NC * _NS
_CH = 128
_NCHUNK = 80
_NMAIN = 78
_NP = 10112
_NV = _NP // 8
_RPT = _NP // _NS
_GRID = 2
_VB = _NV // _GRID

_NTAIL = _NW * _NCHUNK - 2500
_TRASH = np.asarray(
    (_N + np.arange(_NTAIL * _CH) % (_NP - _N)).reshape(1, _NTAIL, _CH),
    np.int32)
_TRASH2 = np.broadcast_to(_TRASH, (2, _NTAIL, _CH))


def _mm_body(x_ref, wl_ref, wr_ref, y_ref, xr_ref):
  ycols, xcols = [], []
  for k in range(8):
    xk = x_ref[:, k, :]
    ycols.append(jnp.dot(xk, wl_ref[...],
                         preferred_element_type=jnp.float32))
    xcols.append(jnp.dot(xk, wr_ref[...],
                         preferred_element_type=jnp.float32))
  y_ref[...] = jnp.concatenate(ycols, axis=1)
  xr_ref[...] = jnp.concatenate(xcols, axis=1)


_mm1 = pl.pallas_call(
    _mm_body,
    grid=(_GRID,),
    in_specs=[
        pl.BlockSpec((_VB, 8, _D), lambda i: (i, 0, 0)),
        pl.BlockSpec((_D, _H), lambda i: (0, 0)),
        pl.BlockSpec((_D, _H), lambda i: (0, 0)),
    ],
    out_specs=[
        pl.BlockSpec((_VB, 128), lambda i: (i, 0)),
        pl.BlockSpec((_VB, 128), lambda i: (i, 0)),
    ],
    out_shape=[
        jax.ShapeDtypeStruct((_NV, 128), jnp.float32),
        jax.ShapeDtypeStruct((_NV, 128), jnp.float32),
    ],
)


_DEPTH = 8


def _load_indices(edge3, pad64, row, buf, wid):
  pltpu.sync_copy(edge3.at[row, pl.ds(wid * _NMAIN, _NMAIN)],
                  buf.at[pl.ds(0, _NMAIN)])
  pltpu.sync_copy(pad64.at[row, pl.ds(2 * wid, 2)],
                  buf.at[pl.ds(_NMAIN, 2)])


def _edge_loop(table, src_v, rows, gsems, scatter):
  for p in range(_DEPTH):
    pltpu.async_copy(table.at[src_v.at[p]], rows[p], gsems[p])

  def group(i, carry):
    j = _DEPTH * i
    descs = []
    for p in range(_DEPTH):
      pltpu.make_async_copy(table.at[src_v.at[j + p]], rows[p],
                            gsems[p]).wait()
      descs.append(scatter(rows[p], j + p, p))
    for p in range(_DEPTH):
      descs[p].wait()
      pltpu.async_copy(table.at[src_v.at[j + _DEPTH + p]], rows[p], gsems[p])
    return carry

  lax.fori_loop(0, _NCHUNK // _DEPTH - 1, group, 0)
  j = _NCHUNK - _DEPTH
  for p in range(_DEPTH):
    pltpu.make_async_copy(table.at[src_v.at[j + p]], rows[p], gsems[p]).wait()
    scatter(rows[p], j + p, p).wait()


def _sc_l1_body(table, edge3, pad64, zeros, ones, agg_o, deg_o,
                src_v, dst_v, r0_v, r1_v, r2_v, r3_v, r4_v, r5_v, r6_v,
                r7_v, ones_v, acc, dega,
                g0, g1, g2, g3, g4, g5, g6, g7,
                s0, s1, s2, s3, s4, s5, s6, s7, dsem):
  cid = lax.axis_index("c")
  sid = lax.axis_index("s")
  wid = sid * _NC + cid
  n0 = sid * _RPT
  rows = [r0_v, r1_v, r2_v, r3_v, r4_v, r5_v, r6_v, r7_v]
  gsems = [g0, g1, g2, g3, g4, g5, g6, g7]
  ssems = [s0, s1, s2, s3, s4, s5, s6, s7]
  pltpu.sync_copy(zeros.at[pl.ds(n0, _RPT)], acc.at[pl.ds(n0, _RPT)])
  pltpu.sync_copy(zeros.at[pl.ds(n0, _RPT)], dega.at[pl.ds(n0, _RPT)])
  pltpu.sync_copy(ones, ones_v)
  _load_indices(edge3, pad64, 0, src_v, wid)
  _load_indices(edge3, pad64, 1, dst_v, wid)
  plsc.subcore_barrier()

  def scatter(rows_buf, j, p):
    d = pltpu.async_copy(rows_buf, acc.at[dst_v.at[j]], ssems[p], add=True)
    pltpu.async_copy(ones_v, dega.at[dst_v.at[j]], dsem, add=True)
    return d

  _edge_loop(table, src_v, rows, gsems, scatter)

  def drain(i, carry):
    pltpu.make_async_copy(ones, ones_v, dsem).wait()
    return carry

  lax.fori_loop(0, _NCHUNK, drain, 0)
  plsc.subcore_barrier()
  pltpu.sync_copy(acc.at[pl.ds(n0, _RPT)], agg_o.at[cid, pl.ds(n0, _RPT)])
  pltpu.sync_copy(dega.at[pl.ds(n0, _RPT)], deg_o.at[cid, pl.ds(n0, _RPT)])


def _sc_l2_body(table, edge3, pad64, zeros, agg_o,
                src_v, dst_v, r0_v, r1_v, r2_v, r3_v, r4_v, r5_v, r6_v,
                r7_v, acc,
                g0, g1, g2, g3, g4, g5, g6, g7,
                s0, s1, s2, s3, s4, s5, s6, s7):
  cid = lax.axis_index("c")
  sid = lax.axis_index("s")
  wid = sid * _NC + cid
  n0 = sid * _RPT
  rows = [r0_v, r1_v, r2_v, r3_v, r4_v, r5_v, r6_v, r7_v]
  gsems = [g0, g1, g2, g3, g4, g5, g6, g7]
  ssems = [s0, s1, s2, s3, s4, s5, s6, s7]
  pltpu.sync_copy(zeros.at[pl.ds(n0, _RPT)], acc.at[pl.ds(n0, _RPT)])
  _load_indices(edge3, pad64, 0, src_v, wid)
  _load_indices(edge3, pad64, 1, dst_v, wid)
  plsc.subcore_barrier()

  def scatter(rows_buf, j, p):
    return pltpu.async_copy(rows_buf, acc.at[dst_v.at[j]], ssems[p],
                            add=True)

  _edge_loop(table, src_v, rows, gsems, scatter)
  plsc.subcore_barrier()
  pltpu.sync_copy(acc.at[pl.ds(n0, _RPT)], agg_o.at[cid, pl.ds(n0, _RPT)])


_sc_mesh = plsc.VectorSubcoreMesh(core_axis_name="c", subcore_axis_name="s")
_sc_params = pltpu.CompilerParams(use_tc_tiling_on_sc=False)

_sc_l1 = pl.kernel(
    _sc_l1_body,
    compiler_params=_sc_params,
    out_type=[
        jax.ShapeDtypeStruct((_NC, _NP, _H), jnp.float32),
        jax.ShapeDtypeStruct((_NC, _NP, _H), jnp.float32),
    ],
    mesh=_sc_mesh,
    scratch_types=[
        pltpu.VMEM((_NCHUNK, _CH), jnp.int32),
        pltpu.VMEM((_NCHUNK, _CH), jnp.int32),
    ] + [pltpu.VMEM((_CH, _H), jnp.float32)] * 9 + [
        pltpu.VMEM_SHARED((_NP, _H), jnp.float32),
        pltpu.VMEM_SHARED((_NP, _H), jnp.float32),
    ] + [pltpu.SemaphoreType.DMA] * 17,
)

_sc_l2 = pl.kernel(
    _sc_l2_body,
    compiler_params=_sc_params,
    out_type=jax.ShapeDtypeStruct((_NC, _NP, _H), jnp.float32),
    mesh=_sc_mesh,
    scratch_types=[
        pltpu.VMEM((_NCHUNK, _CH), jnp.int32),
        pltpu.VMEM((_NCHUNK, _CH), jnp.int32),
    ] + [pltpu.VMEM((_CH, _H), jnp.float32)] * 8 + [
        pltpu.VMEM_SHARED((_NP, _H), jnp.float32),
    ] + [pltpu.SemaphoreType.DMA] * 16,
)


def _tc2_body(aggp, degp, xr, b1, h_o):
  agg = aggp[0] + aggp[1]
  deg = jnp.maximum(degp[0] + degp[1], 1.0)
  h_o[...] = jnp.maximum(agg / deg + b1[...] + xr[...], 0.0)


_tc2 = pl.pallas_call(
    _tc2_body,
    grid=(_GRID,),
    in_specs=[
        pl.BlockSpec((_NC, _VB, 128), lambda i: (0, i, 0)),
        pl.BlockSpec((_NC, _VB, 128), lambda i: (0, i, 0)),
        pl.BlockSpec((_VB, 128), lambda i: (i, 0)),
        pl.BlockSpec((1, 128), lambda i: (0, 0)),
    ],
    out_specs=pl.BlockSpec((_VB, 128), lambda i: (i, 0)),
    out_shape=jax.ShapeDtypeStruct((_NV, 128), jnp.float32),
)


def _tc3_body(aggp, degp, hv, bdwl2, bdwr2, b2, o_ref):
  mean2 = (aggp[0] + aggp[1]) / jnp.maximum(degp[0] + degp[1], 1.0)
  z = (jnp.dot(mean2, bdwl2[...], preferred_element_type=jnp.float32)
       + jnp.dot(hv[...], bdwr2[...], preferred_element_type=jnp.float32)
       + b2[...])
  outs = []
  for g in range(8):
    zg = z[:, _C * g:_C * (g + 1)]
    m = jnp.max(zg, axis=1, keepdims=True)
    s = jnp.sum(jnp.exp(zg - m), axis=1, keepdims=True)
    outs.append(zg - m - jnp.log(s))
  o_ref[...] = jnp.concatenate(outs, axis=1)


_tc3 = pl.pallas_call(
    _tc3_body,
    grid=(_GRID,),
    in_specs=[
        pl.BlockSpec((_NC, _VB, 128), lambda i: (0, i, 0)),
        pl.BlockSpec((_NC, _VB, 128), lambda i: (0, i, 0)),
        pl.BlockSpec((_VB, 128), lambda i: (i, 0)),
        pl.BlockSpec((128, 8 * _C), lambda i: (0, 0)),
        pl.BlockSpec((128, 8 * _C), lambda i: (0, 0)),
        pl.BlockSpec((1, 8 * _C), lambda i: (0, 0)),
    ],
    out_specs=pl.BlockSpec((_VB, 8 * _C), lambda i: (i, 0)),
    out_shape=jax.ShapeDtypeStruct((_NV, 8 * _C), jnp.float32),
)


def kernel(x, edge_index, W_l1, b_l1, W_r1, W_l2, b_l2, W_r2):
  x3 = jnp.pad(x, ((0, _NP - _N), (0, 0))).reshape(_NV, 8, _D)
  edge3 = edge_index.reshape(2, _E // _CH, _CH)
  tail = edge_index[:, _NMAIN * _NW * _CH:].reshape(2, -1, _CH)
  pad64 = jnp.concatenate([tail, jnp.asarray(_TRASH2)], axis=1)
  zeros = jnp.zeros((_NP, _H), jnp.float32)
  ones = jnp.ones((_CH, _H), jnp.float32)
  eye8 = jnp.eye(8, dtype=jnp.float32)
  bdwl2 = jnp.kron(eye8, W_l2)
  bdwr2 = jnp.kron(eye8, W_r2)
  b1t = jnp.tile(b_l1, 8).reshape(1, 128)
  b2t = jnp.tile(b_l2, 8).reshape(1, 8 * _C)

  y1v, xrv = _mm1(x3, W_l1, W_r1)
  aggp, degp = _sc_l1(y1v.reshape(_NP, _H), edge3, pad64, zeros, ones)
  aggv = aggp.reshape(_NC, _NV, 128)
  degv = degp.reshape(_NC, _NV, 128)
  hv = _tc2(aggv, degv, xrv, b1t)
  agg2p = _sc_l2(hv.reshape(_NP, _H), edge3, pad64, zeros)
  outv = _tc3(agg2p.reshape(_NC, _NV, 128), degv, hv, bdwl2, bdwr2, b2t)
  return outv.reshape(_NP, _C)[:_N]


# speedup vs baseline: 1.6690x; 1.0010x over previous
import jax
import jax.numpy as jnp
import numpy as np
from jax import lax
from jax.experimental import pallas as pl
from jax.experimental.pallas import tpu as pltpu
from jax.experimental.pallas import tpu_sc as plsc

_N = 10000
_E = 320000
_D = 128
_H = 16
_C = 40

_NC, _NS = 2, 16
_NW = _NC * _NS
_CH = 128
_NCHUNK = 80
_NMAIN = 78
_NP = 10112
_NV = _NP // 8
_RPT = _NP // _NS
_GRID = 2
_VB = _NV // _GRID

_NTAIL = _NW * _NCHUNK - 2500
_TRASH = np.asarray(
    (_N + np.arange(_NTAIL * _CH) % (_NP - _N)).reshape(1, _NTAIL, _CH),
    np.int32)
_TRASH2 = np.broadcast_to(_TRASH, (2, _NTAIL, _CH))


def _mm_body(x_ref, wl_ref, wr_ref, y_ref, xr_ref):
  ycols, xcols = [], []
  for k in range(8):
    xk = x_ref[:, k, :]
    ycols.append(jnp.dot(xk, wl_ref[...],
                         preferred_element_type=jnp.float32))
    xcols.append(jnp.dot(xk, wr_ref[...],
                         preferred_element_type=jnp.float32))
  y_ref[...] = jnp.concatenate(ycols, axis=1)
  xr_ref[...] = jnp.concatenate(xcols, axis=1)


_mm1 = pl.pallas_call(
    _mm_body,
    grid=(_GRID,),
    in_specs=[
        pl.BlockSpec((_VB, 8, _D), lambda i: (i, 0, 0)),
        pl.BlockSpec((_D, _H), lambda i: (0, 0)),
        pl.BlockSpec((_D, _H), lambda i: (0, 0)),
    ],
    out_specs=[
        pl.BlockSpec((_VB, 128), lambda i: (i, 0)),
        pl.BlockSpec((_VB, 128), lambda i: (i, 0)),
    ],
    out_shape=[
        jax.ShapeDtypeStruct((_NV, 128), jnp.float32),
        jax.ShapeDtypeStruct((_NV, 128), jnp.float32),
    ],
)


_DEPTH = 8


def _load_indices(edge3, pad64, row, buf, wid):
  pltpu.sync_copy(edge3.at[row, pl.ds(wid * _NMAIN, _NMAIN)],
                  buf.at[pl.ds(0, _NMAIN)])
  pltpu.sync_copy(pad64.at[row, pl.ds(2 * wid, 2)],
                  buf.at[pl.ds(_NMAIN, 2)])


def _edge_loop(table, src_v, rows, gsems, scatter):
  for p in range(_DEPTH):
    pltpu.async_copy(table.at[src_v.at[p]], rows[p], gsems[p])

  def group(i, carry):
    j = _DEPTH * i
    descs = []
    for p in range(_DEPTH):
      pltpu.make_async_copy(table.at[src_v.at[j + p]], rows[p],
                            gsems[p]).wait()
      descs.append(scatter(rows[p], j + p, p))
    for p in range(_DEPTH):
      descs[p].wait()
      pltpu.async_copy(table.at[src_v.at[j + _DEPTH + p]], rows[p], gsems[p])
    return carry

  lax.fori_loop(0, _NCHUNK // _DEPTH - 1, group, 0)
  j = _NCHUNK - _DEPTH
  for p in range(_DEPTH):
    pltpu.make_async_copy(table.at[src_v.at[j + p]], rows[p], gsems[p]).wait()
    scatter(rows[p], j + p, p).wait()


def _sc_l1_body(table, edge3, pad64, zeros, ones, agg_o, deg_o,
                src_v, dst_v, *bufs):
  ones_v, acc, dega = bufs[8], bufs[9], bufs[10]
  rows = list(bufs[0:8])
  gsems = list(bufs[11:19])
  ssems = list(bufs[19:27])
  dsem = bufs[27]
  cid = lax.axis_index("c")
  sid = lax.axis_index("s")
  wid = sid * _NC + cid
  n0 = sid * _RPT
  pltpu.sync_copy(zeros.at[pl.ds(n0, _RPT)], acc.at[pl.ds(n0, _RPT)])
  pltpu.sync_copy(zeros.at[pl.ds(n0, _RPT)], dega.at[pl.ds(n0, _RPT)])
  pltpu.sync_copy(ones, ones_v)
  _load_indices(edge3, pad64, 0, src_v, wid)
  _load_indices(edge3, pad64, 1, dst_v, wid)
  plsc.subcore_barrier()

  def scatter(rows_buf, j, p):
    d = pltpu.async_copy(rows_buf, acc.at[dst_v.at[j]], ssems[p], add=True)
    pltpu.async_copy(ones_v, dega.at[dst_v.at[j]], dsem, add=True)
    return d

  _edge_loop(table, src_v, rows, gsems, scatter)

  def drain(i, carry):
    pltpu.make_async_copy(ones, ones_v, dsem).wait()
    return carry

  lax.fori_loop(0, _NCHUNK, drain, 0)
  plsc.subcore_barrier()
  pltpu.sync_copy(acc.at[pl.ds(n0, _RPT)], agg_o.at[cid, pl.ds(n0, _RPT)])
  pltpu.sync_copy(dega.at[pl.ds(n0, _RPT)], deg_o.at[cid, pl.ds(n0, _RPT)])


def _sc_l2_body(table, edge3, pad64, zeros, agg_o,
                src_v, dst_v, *bufs):
  acc = bufs[8]
  rows = list(bufs[0:8])
  gsems = list(bufs[9:17])
  ssems = list(bufs[17:25])
  cid = lax.axis_index("c")
  sid = lax.axis_index("s")
  wid = sid * _NC + cid
  n0 = sid * _RPT
  pltpu.sync_copy(zeros.at[pl.ds(n0, _RPT)], acc.at[pl.ds(n0, _RPT)])
  _load_indices(edge3, pad64, 0, src_v, wid)
  _load_indices(edge3, pad64, 1, dst_v, wid)
  plsc.subcore_barrier()

  def scatter(rows_buf, j, p):
    return pltpu.async_copy(rows_buf, acc.at[dst_v.at[j]], ssems[p],
                            add=True)

  _edge_loop(table, src_v, rows, gsems, scatter)
  plsc.subcore_barrier()
  pltpu.sync_copy(acc.at[pl.ds(n0, _RPT)], agg_o.at[cid, pl.ds(n0, _RPT)])


_sc_mesh = plsc.VectorSubcoreMesh(core_axis_name="c", subcore_axis_name="s")
_sc_params = pltpu.CompilerParams(use_tc_tiling_on_sc=False)

_sc_l1 = pl.kernel(
    _sc_l1_body,
    compiler_params=_sc_params,
    out_type=[
        jax.ShapeDtypeStruct((_NC, _NP, _H), jnp.float32),
        jax.ShapeDtypeStruct((_NC, _NP, _H), jnp.float32),
    ],
    mesh=_sc_mesh,
    scratch_types=[
        pltpu.VMEM((_NCHUNK, _CH), jnp.int32),
        pltpu.VMEM((_NCHUNK, _CH), jnp.int32),
    ] + [pltpu.VMEM((_CH, _H), jnp.float32)] * 9 + [
        pltpu.VMEM_SHARED((_NP, _H), jnp.float32),
        pltpu.VMEM_SHARED((_NP, _H), jnp.float32),
    ] + [pltpu.SemaphoreType.DMA] * 17,
)

_sc_l2 = pl.kernel(
    _sc_l2_body,
    compiler_params=_sc_params,
    out_type=jax.ShapeDtypeStruct((_NC, _NP, _H), jnp.float32),
    mesh=_sc_mesh,
    scratch_types=[
        pltpu.VMEM((_NCHUNK, _CH), jnp.int32),
        pltpu.VMEM((_NCHUNK, _CH), jnp.int32),
    ] + [pltpu.VMEM((_CH, _H), jnp.float32)] * 8 + [
        pltpu.VMEM_SHARED((_NP, _H), jnp.float32),
    ] + [pltpu.SemaphoreType.DMA] * 16,
)


def _tc2_body(aggp, degp, xr, b1, h_o):
  agg = aggp[0] + aggp[1]
  deg = jnp.maximum(degp[0] + degp[1], 1.0)
  h_o[...] = jnp.maximum(agg / deg + b1[...] + xr[...], 0.0)


_tc2 = pl.pallas_call(
    _tc2_body,
    grid=(_GRID,),
    in_specs=[
        pl.BlockSpec((_NC, _VB, 128), lambda i: (0, i, 0)),
        pl.BlockSpec((_NC, _VB, 128), lambda i: (0, i, 0)),
        pl.BlockSpec((_VB, 128), lambda i: (i, 0)),
        pl.BlockSpec((1, 128), lambda i: (0, 0)),
    ],
    out_specs=pl.BlockSpec((_VB, 128), lambda i: (i, 0)),
    out_shape=jax.ShapeDtypeStruct((_NV, 128), jnp.float32),
)


def _tc3_body(aggp, degp, hv, bdwl2, bdwr2, b2, o_ref):
  mean2 = (aggp[0] + aggp[1]) / jnp.maximum(degp[0] + degp[1], 1.0)
  z = (jnp.dot(mean2, bdwl2[...], preferred_element_type=jnp.float32)
       + jnp.dot(hv[...], bdwr2[...], preferred_element_type=jnp.float32)
       + b2[...])
  outs = []
  for g in range(8):
    zg = z[:, _C * g:_C * (g + 1)]
    m = jnp.max(zg, axis=1, keepdims=True)
    s = jnp.sum(jnp.exp(zg - m), axis=1, keepdims=True)
    outs.append(zg - m - jnp.log(s))
  o_ref[...] = jnp.concatenate(outs, axis=1)


_tc3 = pl.pallas_call(
    _tc3_body,
    grid=(_GRID,),
    in_specs=[
        pl.BlockSpec((_NC, _VB, 128), lambda i: (0, i, 0)),
        pl.BlockSpec((_NC, _VB, 128), lambda i: (0, i, 0)),
        pl.BlockSpec((_VB, 128), lambda i: (i, 0)),
        pl.BlockSpec((128, 8 * _C), lambda i: (0, 0)),
        pl.BlockSpec((128, 8 * _C), lambda i: (0, 0)),
        pl.BlockSpec((1, 8 * _C), lambda i: (0, 0)),
    ],
    out_specs=pl.BlockSpec((_VB, 8 * _C), lambda i: (i, 0)),
    out_shape=jax.ShapeDtypeStruct((_NV, 8 * _C), jnp.float32),
)


def kernel(x, edge_index, W_l1, b_l1, W_r1, W_l2, b_l2, W_r2):
  x3 = jnp.pad(x, ((0, _NP - _N), (0, 0))).reshape(_NV, 8, _D)
  edge3 = edge_index.reshape(2, _E // _CH, _CH)
  tail = edge_index[:, _NMAIN * _NW * _CH:].reshape(2, -1, _CH)
  pad64 = jnp.concatenate([tail, jnp.asarray(_TRASH2)], axis=1)
  zeros = jnp.zeros((_NP, _H), jnp.float32)
  ones = jnp.ones((_CH, _H), jnp.float32)
  eye8 = jnp.eye(8, dtype=jnp.float32)
  bdwl2 = jnp.kron(eye8, W_l2)
  bdwr2 = jnp.kron(eye8, W_r2)
  b1t = jnp.tile(b_l1, 8).reshape(1, 128)
  b2t = jnp.tile(b_l2, 8).reshape(1, 8 * _C)

  y1v, xrv = _mm1(x3, W_l1, W_r1)
  aggp, degp = _sc_l1(y1v.reshape(_NP, _H), edge3, pad64, zeros, ones)
  aggv = aggp.reshape(_NC, _NV, 128)
  degv = degp.reshape(_NC, _NV, 128)
  hv = _tc2(aggv, degv, xrv, b1t)
  agg2p = _sc_l2(hv.reshape(_NP, _H), edge3, pad64, zeros)
  outv = _tc3(agg2p.reshape(_NC, _NV, 128), degv, hv, bdwl2, bdwr2, b2t)
  return outv.reshape(_NP, _C)[:_N]


# speedup vs baseline: 1.7763x; 1.0643x over previous
import jax
import jax.numpy as jnp
import numpy as np
from jax import lax
from jax.experimental import pallas as pl
from jax.experimental.pallas import tpu as pltpu
from jax.experimental.pallas import tpu_sc as plsc

_N = 10000
_E = 320000
_D = 128
_H = 16
_C = 40

_NC, _NS = 2, 16
_NW = _NC * _NS
_CH = 128
_NCHUNK = 80
_NMAIN = 78
_NP = 10112
_NV = _NP // 8
_RPT = _NP // _NS
_GRID = 2
_VB = _NV // _GRID

_NTAIL = _NW * _NCHUNK - 2500
_TRASH = np.asarray(
    (_N + np.arange(_NTAIL * _CH) % (_NP - _N)).reshape(1, _NTAIL, _CH),
    np.int32)
_TRASH2 = np.broadcast_to(_TRASH, (2, _NTAIL, _CH))


def _mm_body(x_ref, wl_ref, wr_ref, y_ref, xr_ref):
  ycols, xcols = [], []
  for k in range(8):
    xk = x_ref[:, k, :]
    ycols.append(jnp.dot(xk, wl_ref[...],
                         preferred_element_type=jnp.float32))
    xcols.append(jnp.dot(xk, wr_ref[...],
                         preferred_element_type=jnp.float32))
  y_ref[...] = jnp.concatenate(ycols, axis=1)
  xr_ref[...] = jnp.concatenate(xcols, axis=1)


_mm1 = pl.pallas_call(
    _mm_body,
    grid=(_GRID,),
    in_specs=[
        pl.BlockSpec((_VB, 8, _D), lambda i: (i, 0, 0)),
        pl.BlockSpec((_D, _H), lambda i: (0, 0)),
        pl.BlockSpec((_D, _H), lambda i: (0, 0)),
    ],
    out_specs=[
        pl.BlockSpec((_VB, 128), lambda i: (i, 0)),
        pl.BlockSpec((_VB, 128), lambda i: (i, 0)),
    ],
    out_shape=[
        jax.ShapeDtypeStruct((_NV, 128), jnp.float32),
        jax.ShapeDtypeStruct((_NV, 128), jnp.float32),
    ],
)


_DEPTH = 8


def _load_indices(edge3, pad64, row, buf, wid):
  pltpu.sync_copy(edge3.at[row, pl.ds(wid * _NMAIN, _NMAIN)],
                  buf.at[pl.ds(0, _NMAIN)])
  pltpu.sync_copy(pad64.at[row, pl.ds(2 * wid, 2)],
                  buf.at[pl.ds(_NMAIN, 2)])


def _edge_loop(table, src_v, rows, gsems, scatter):
  for p in range(_DEPTH):
    pltpu.async_copy(table.at[src_v.at[p]], rows[p], gsems[p])

  def group(i, carry):
    j = _DEPTH * i
    descs = []
    for p in range(_DEPTH):
      pltpu.make_async_copy(table.at[src_v.at[j + p]], rows[p],
                            gsems[p]).wait()
      descs.append(scatter(rows[p], j + p, p))
    for p in range(_DEPTH):
      descs[p].wait()
      pltpu.async_copy(table.at[src_v.at[j + _DEPTH + p]], rows[p], gsems[p])
    return carry

  lax.fori_loop(0, _NCHUNK // _DEPTH - 1, group, 0)
  j = _NCHUNK - _DEPTH
  for p in range(_DEPTH):
    pltpu.make_async_copy(table.at[src_v.at[j + p]], rows[p], gsems[p]).wait()
    scatter(rows[p], j + p, p).wait()


def _sc_l1_body(table, edge3, pad64, zeros, ones, agg_o, deg_o,
                src_v, dst_v, *bufs):
  ones_v, acc, dega = bufs[8], bufs[9], bufs[10]
  rows = list(bufs[0:8])
  gsems = list(bufs[11:19])
  ssems = list(bufs[19:27])
  dsem = bufs[27]
  cid = lax.axis_index("c")
  sid = lax.axis_index("s")
  wid = sid * _NC + cid
  n0 = sid * _RPT
  pltpu.sync_copy(zeros.at[pl.ds(n0, _RPT)], acc.at[pl.ds(n0, _RPT)])
  pltpu.sync_copy(zeros.at[pl.ds(n0, _RPT)], dega.at[pl.ds(n0, _RPT)])
  pltpu.sync_copy(ones, ones_v)
  _load_indices(edge3, pad64, 0, src_v, wid)
  _load_indices(edge3, pad64, 1, dst_v, wid)
  plsc.subcore_barrier()

  def scatter(rows_buf, j, p):
    d = pltpu.async_copy(rows_buf, acc.at[dst_v.at[j]], ssems[p], add=True)
    pltpu.async_copy(ones_v, dega.at[dst_v.at[j]], dsem, add=True)
    return d

  _edge_loop(table, src_v, rows, gsems, scatter)

  def drain(i, carry):
    pltpu.make_async_copy(ones, ones_v, dsem).wait()
    return carry

  lax.fori_loop(0, _NCHUNK, drain, 0)
  plsc.subcore_barrier()
  pltpu.sync_copy(acc.at[pl.ds(n0, _RPT)], agg_o.at[cid, pl.ds(n0, _RPT)])
  pltpu.sync_copy(dega.at[pl.ds(n0, _RPT)], deg_o.at[cid, pl.ds(n0, _RPT)])


def _sc_l2_body(table, edge3, pad64, zeros, agg_o,
                src_v, dst_v, *bufs):
  acc = bufs[8]
  rows = list(bufs[0:8])
  gsems = list(bufs[9:17])
  ssems = list(bufs[17:25])
  cid = lax.axis_index("c")
  sid = lax.axis_index("s")
  wid = sid * _NC + cid
  n0 = sid * _RPT
  pltpu.sync_copy(zeros.at[pl.ds(n0, _RPT)], acc.at[pl.ds(n0, _RPT)])
  _load_indices(edge3, pad64, 0, src_v, wid)
  _load_indices(edge3, pad64, 1, dst_v, wid)
  plsc.subcore_barrier()

  def scatter(rows_buf, j, p):
    return pltpu.async_copy(rows_buf, acc.at[dst_v.at[j]], ssems[p],
                            add=True)

  _edge_loop(table, src_v, rows, gsems, scatter)
  plsc.subcore_barrier()
  pltpu.sync_copy(acc.at[pl.ds(n0, _RPT)], agg_o.at[cid, pl.ds(n0, _RPT)])


_sc_mesh = plsc.VectorSubcoreMesh(core_axis_name="c", subcore_axis_name="s")
_sc_params = pltpu.CompilerParams(use_tc_tiling_on_sc=False)

_sc_l1 = pl.kernel(
    _sc_l1_body,
    compiler_params=_sc_params,
    out_type=[
        jax.ShapeDtypeStruct((_NC, _NP, _H), jnp.float32),
        jax.ShapeDtypeStruct((_NC, _NP, _H), jnp.float32),
    ],
    mesh=_sc_mesh,
    scratch_types=[
        pltpu.VMEM((_NCHUNK, _CH), jnp.int32),
        pltpu.VMEM((_NCHUNK, _CH), jnp.int32),
    ] + [pltpu.VMEM((_CH, _H), jnp.float32)] * 9 + [
        pltpu.VMEM_SHARED((_NP, _H), jnp.float32),
        pltpu.VMEM_SHARED((_NP, _H), jnp.float32),
    ] + [pltpu.SemaphoreType.DMA] * 17,
)

_sc_l2 = pl.kernel(
    _sc_l2_body,
    compiler_params=_sc_params,
    out_type=jax.ShapeDtypeStruct((_NC, _NP, _H), jnp.float32),
    mesh=_sc_mesh,
    scratch_types=[
        pltpu.VMEM((_NCHUNK, _CH), jnp.int32),
        pltpu.VMEM((_NCHUNK, _CH), jnp.int32),
    ] + [pltpu.VMEM((_CH, _H), jnp.float32)] * 8 + [
        pltpu.VMEM_SHARED((_NP, _H), jnp.float32),
    ] + [pltpu.SemaphoreType.DMA] * 16,
)


def _tc2_body(aggp, degp, xr, b1, h_o):
  agg = aggp[0] + aggp[1]
  deg = jnp.maximum(degp[0] + degp[1], 1.0)
  h_o[...] = jnp.maximum(agg / deg + b1[...] + xr[...], 0.0)


_tc2 = pl.pallas_call(
    _tc2_body,
    grid=(_GRID,),
    in_specs=[
        pl.BlockSpec((_NC, _VB, 128), lambda i: (0, i, 0)),
        pl.BlockSpec((_NC, _VB, 128), lambda i: (0, i, 0)),
        pl.BlockSpec((_VB, 128), lambda i: (i, 0)),
        pl.BlockSpec((1, 128), lambda i: (0, 0)),
    ],
    out_specs=pl.BlockSpec((_VB, 128), lambda i: (i, 0)),
    out_shape=jax.ShapeDtypeStruct((_NV, 128), jnp.float32),
)


def _tc3_body(aggp, degp, hv, bdwl2, bdwr2, b2, gsum, o_ref):
  mean2 = (aggp[0] + aggp[1]) / jnp.maximum(degp[0] + degp[1], 1.0)
  z = (jnp.dot(mean2, bdwl2[...], preferred_element_type=jnp.float32)
       + jnp.dot(hv[...], bdwr2[...], preferred_element_type=jnp.float32)
       + b2[...])
  m = jnp.max(z, axis=1, keepdims=True)
  e = jnp.exp(z - m)
  s = jnp.dot(e, gsum[...], preferred_element_type=jnp.float32)
  o_ref[...] = z - m - jnp.log(s)


_tc3 = pl.pallas_call(
    _tc3_body,
    grid=(_GRID,),
    in_specs=[
        pl.BlockSpec((_NC, _VB, 128), lambda i: (0, i, 0)),
        pl.BlockSpec((_NC, _VB, 128), lambda i: (0, i, 0)),
        pl.BlockSpec((_VB, 128), lambda i: (i, 0)),
        pl.BlockSpec((128, 8 * _C), lambda i: (0, 0)),
        pl.BlockSpec((128, 8 * _C), lambda i: (0, 0)),
        pl.BlockSpec((1, 8 * _C), lambda i: (0, 0)),
        pl.BlockSpec((8 * _C, 8 * _C), lambda i: (0, 0)),
    ],
    out_specs=pl.BlockSpec((_VB, 8 * _C), lambda i: (i, 0)),
    out_shape=jax.ShapeDtypeStruct((_NV, 8 * _C), jnp.float32),
)


def kernel(x, edge_index, W_l1, b_l1, W_r1, W_l2, b_l2, W_r2):
  x3 = jnp.pad(x, ((0, _NP - _N), (0, 0))).reshape(_NV, 8, _D)
  edge3 = edge_index.reshape(2, _E // _CH, _CH)
  tail = edge_index[:, _NMAIN * _NW * _CH:].reshape(2, -1, _CH)
  pad64 = jnp.concatenate([tail, jnp.asarray(_TRASH2)], axis=1)
  zeros = jnp.zeros((_NP, _H), jnp.float32)
  ones = jnp.ones((_CH, _H), jnp.float32)
  eye8 = jnp.eye(8, dtype=jnp.float32)
  bdwl2 = jnp.kron(eye8, W_l2)
  bdwr2 = jnp.kron(eye8, W_r2)
  b1t = jnp.tile(b_l1, 8).reshape(1, 128)
  b2t = jnp.tile(b_l2, 8).reshape(1, 8 * _C)

  y1v, xrv = _mm1(x3, W_l1, W_r1)
  aggp, degp = _sc_l1(y1v.reshape(_NP, _H), edge3, pad64, zeros, ones)
  aggv = aggp.reshape(_NC, _NV, 128)
  degv = degp.reshape(_NC, _NV, 128)
  hv = _tc2(aggv, degv, xrv, b1t)
  agg2p = _sc_l2(hv.reshape(_NP, _H), edge3, pad64, zeros)
  gsum = jnp.kron(eye8, jnp.ones((_C, _C), jnp.float32))
  outv = _tc3(agg2p.reshape(_NC, _NV, 128), degv, hv, bdwl2, bdwr2, b2t,
              gsum)
  return outv.reshape(_NP, _C)[:_N]


# speedup vs baseline: 1.7769x; 1.0004x over previous
import jax
import jax.numpy as jnp
import numpy as np
from jax import lax
from jax.experimental import pallas as pl
from jax.experimental.pallas import tpu as pltpu
from jax.experimental.pallas import tpu_sc as plsc

_N = 10000
_E = 320000
_D = 128
_H = 16
_C = 40

_NC, _NS = 2, 16
_NW = _NC * _NS
_CH = 128
_NCHUNK = 80
_NMAIN = 78
_NP = 10112
_NV = _NP // 8
_RPT = _NP // _NS
_GRID = 2
_VB = _NV // _GRID

_NTAIL = _NW * _NCHUNK - 2500
_TRASH = np.asarray(
    (_N + np.arange(_NTAIL * _CH) % (_NP - _N)).reshape(1, _NTAIL, _CH),
    np.int32)
_TRASH2 = np.broadcast_to(_TRASH, (2, _NTAIL, _CH))


def _mm_body(x_ref, wl_ref, wr_ref, y_ref, xr_ref):
  ycols, xcols = [], []
  for k in range(8):
    xk = x_ref[:, k, :]
    ycols.append(jnp.dot(xk, wl_ref[...],
                         preferred_element_type=jnp.float32))
    xcols.append(jnp.dot(xk, wr_ref[...],
                         preferred_element_type=jnp.float32))
  y_ref[...] = jnp.concatenate(ycols, axis=1)
  xr_ref[...] = jnp.concatenate(xcols, axis=1)


_mm1 = pl.pallas_call(
    _mm_body,
    grid=(_GRID,),
    in_specs=[
        pl.BlockSpec((_VB, 8, _D), lambda i: (i, 0, 0)),
        pl.BlockSpec((_D, _H), lambda i: (0, 0)),
        pl.BlockSpec((_D, _H), lambda i: (0, 0)),
    ],
    out_specs=[
        pl.BlockSpec((_VB, 128), lambda i: (i, 0)),
        pl.BlockSpec((_VB, 128), lambda i: (i, 0)),
    ],
    out_shape=[
        jax.ShapeDtypeStruct((_NV, 128), jnp.float32),
        jax.ShapeDtypeStruct((_NV, 128), jnp.float32),
    ],
)


_DEPTH = 8


def _load_indices(edge3, pad64, row, buf, wid):
  pltpu.sync_copy(edge3.at[row, pl.ds(wid * _NMAIN, _NMAIN)],
                  buf.at[pl.ds(0, _NMAIN)])
  pltpu.sync_copy(pad64.at[row, pl.ds(2 * wid, 2)],
                  buf.at[pl.ds(_NMAIN, 2)])


def _edge_loop(table, src_v, rows, gsems, scatter):
  for p in range(_DEPTH):
    pltpu.async_copy(table.at[src_v.at[p]], rows[p], gsems[p])

  def group(i, carry):
    j = _DEPTH * i
    descs = []
    for p in range(_DEPTH):
      pltpu.make_async_copy(table.at[src_v.at[j + p]], rows[p],
                            gsems[p]).wait()
      descs.append(scatter(rows[p], j + p, p))
    for p in range(_DEPTH):
      descs[p].wait()
      pltpu.async_copy(table.at[src_v.at[j + _DEPTH + p]], rows[p], gsems[p])
    return carry

  lax.fori_loop(0, _NCHUNK // _DEPTH - 1, group, 0)
  j = _NCHUNK - _DEPTH
  for p in range(_DEPTH):
    pltpu.make_async_copy(table.at[src_v.at[j + p]], rows[p], gsems[p]).wait()
    scatter(rows[p], j + p, p).wait()


def _sc_l1_body(table, edge3, pad64, zeros, ones, agg_o, deg_o,
                src_v, dst_v, *bufs):
  ones_v, acc, dega = bufs[8], bufs[9], bufs[10]
  rows = list(bufs[0:8])
  gsems = list(bufs[11:19])
  ssems = list(bufs[19:27])
  dsem = bufs[27]
  cid = lax.axis_index("c")
  sid = lax.axis_index("s")
  wid = sid * _NC + cid
  n0 = sid * _RPT
  pltpu.sync_copy(zeros.at[pl.ds(n0, _RPT)], acc.at[pl.ds(n0, _RPT)])
  pltpu.sync_copy(zeros.at[pl.ds(n0, _RPT)], dega.at[pl.ds(n0, _RPT)])
  pltpu.sync_copy(ones, ones_v)
  _load_indices(edge3, pad64, 0, src_v, wid)
  _load_indices(edge3, pad64, 1, dst_v, wid)
  plsc.subcore_barrier()

  def scatter(rows_buf, j, p):
    d = pltpu.async_copy(rows_buf, acc.at[dst_v.at[j]], ssems[p], add=True)
    pltpu.async_copy(ones_v, dega.at[dst_v.at[j]], dsem, add=True)
    return d

  _edge_loop(table, src_v, rows, gsems, scatter)

  def drain(i, carry):
    pltpu.make_async_copy(ones, ones_v, dsem).wait()
    return carry

  lax.fori_loop(0, _NCHUNK, drain, 0)
  plsc.subcore_barrier()
  pltpu.sync_copy(acc.at[pl.ds(n0, _RPT)], agg_o.at[cid, pl.ds(n0, _RPT)])
  pltpu.sync_copy(dega.at[pl.ds(n0, _RPT)], deg_o.at[cid, pl.ds(n0, _RPT)])


def _sc_l2_body(table, edge3, pad64, zeros, agg_o,
                src_v, dst_v, *bufs):
  acc = bufs[8]
  rows = list(bufs[0:8])
  gsems = list(bufs[9:17])
  ssems = list(bufs[17:25])
  cid = lax.axis_index("c")
  sid = lax.axis_index("s")
  wid = sid * _NC + cid
  n0 = sid * _RPT
  pltpu.sync_copy(zeros.at[pl.ds(n0, _RPT)], acc.at[pl.ds(n0, _RPT)])
  _load_indices(edge3, pad64, 0, src_v, wid)
  _load_indices(edge3, pad64, 1, dst_v, wid)
  plsc.subcore_barrier()

  def scatter(rows_buf, j, p):
    return pltpu.async_copy(rows_buf, acc.at[dst_v.at[j]], ssems[p],
                            add=True)

  _edge_loop(table, src_v, rows, gsems, scatter)
  plsc.subcore_barrier()
  pltpu.sync_copy(acc.at[pl.ds(n0, _RPT)], agg_o.at[cid, pl.ds(n0, _RPT)])


_sc_mesh = plsc.VectorSubcoreMesh(core_axis_name="c", subcore_axis_name="s")
_sc_params = pltpu.CompilerParams(use_tc_tiling_on_sc=False)

_sc_l1 = pl.kernel(
    _sc_l1_body,
    compiler_params=_sc_params,
    out_type=[
        jax.ShapeDtypeStruct((_NC, _NP, _H), jnp.float32),
        jax.ShapeDtypeStruct((_NC, _NP, _H), jnp.float32),
    ],
    mesh=_sc_mesh,
    scratch_types=[
        pltpu.VMEM((_NCHUNK, _CH), jnp.int32),
        pltpu.VMEM((_NCHUNK, _CH), jnp.int32),
    ] + [pltpu.VMEM((_CH, _H), jnp.float32)] * 9 + [
        pltpu.VMEM_SHARED((_NP, _H), jnp.float32),
        pltpu.VMEM_SHARED((_NP, _H), jnp.float32),
    ] + [pltpu.SemaphoreType.DMA] * 17,
)

_sc_l2 = pl.kernel(
    _sc_l2_body,
    compiler_params=_sc_params,
    out_type=jax.ShapeDtypeStruct((_NC, _NP, _H), jnp.float32),
    mesh=_sc_mesh,
    scratch_types=[
        pltpu.VMEM((_NCHUNK, _CH), jnp.int32),
        pltpu.VMEM((_NCHUNK, _CH), jnp.int32),
    ] + [pltpu.VMEM((_CH, _H), jnp.float32)] * 8 + [
        pltpu.VMEM_SHARED((_NP, _H), jnp.float32),
    ] + [pltpu.SemaphoreType.DMA] * 16,
)


def _tc2_body(aggp, degp, xr, b1, h_o):
  agg = aggp[0] + aggp[1]
  deg = jnp.maximum(degp[0] + degp[1], 1.0)
  h_o[...] = jnp.maximum(agg / deg + b1[...] + xr[...], 0.0)


_tc2 = pl.pallas_call(
    _tc2_body,
    grid=(_GRID,),
    in_specs=[
        pl.BlockSpec((_NC, _VB, 128), lambda i: (0, i, 0)),
        pl.BlockSpec((_NC, _VB, 128), lambda i: (0, i, 0)),
        pl.BlockSpec((_VB, 128), lambda i: (i, 0)),
        pl.BlockSpec((1, 128), lambda i: (0, 0)),
    ],
    out_specs=pl.BlockSpec((_VB, 128), lambda i: (i, 0)),
    out_shape=jax.ShapeDtypeStruct((_NV, 128), jnp.float32),
)


def _tc3_body(aggp, degp, hv, bdwl2, bdwr2, b2, gsum, o_ref):
  mean2 = (aggp[0] + aggp[1]) / jnp.maximum(degp[0] + degp[1], 1.0)
  z = (jnp.dot(mean2, bdwl2[...], preferred_element_type=jnp.float32)
       + jnp.dot(hv[...], bdwr2[...], preferred_element_type=jnp.float32)
       + b2[...])
  m = jnp.max(z, axis=1, keepdims=True)
  e = jnp.exp(z - m)
  s = jnp.dot(e, gsum[...], preferred_element_type=jnp.float32)
  o_ref[...] = z - m - jnp.log(s)


_tc3 = pl.pallas_call(
    _tc3_body,
    grid=(_GRID,),
    in_specs=[
        pl.BlockSpec((_NC, _VB, 128), lambda i: (0, i, 0)),
        pl.BlockSpec((_NC, _VB, 128), lambda i: (0, i, 0)),
        pl.BlockSpec((_VB, 128), lambda i: (i, 0)),
        pl.BlockSpec((128, 8 * _C), lambda i: (0, 0)),
        pl.BlockSpec((128, 8 * _C), lambda i: (0, 0)),
        pl.BlockSpec((1, 8 * _C), lambda i: (0, 0)),
        pl.BlockSpec((8 * _C, 8 * _C), lambda i: (0, 0)),
    ],
    out_specs=pl.BlockSpec((_VB, 8 * _C), lambda i: (i, 0)),
    out_shape=jax.ShapeDtypeStruct((_NV, 8 * _C), jnp.float32),
)


def kernel(x, edge_index, W_l1, b_l1, W_r1, W_l2, b_l2, W_r2):
  x3 = jnp.pad(x, ((0, _NP - _N), (0, 0))).reshape(_NV, 8, _D)
  edge3 = edge_index.reshape(2, _E // _CH, _CH)
  tail = edge_index[:, _NMAIN * _NW * _CH:].reshape(2, -1, _CH)
  pad64 = jnp.concatenate([tail, jnp.asarray(_TRASH2)], axis=1)
  zeros = jnp.zeros((_NP, _H), jnp.float32)
  ones = jnp.ones((_CH, _H), jnp.float32)
  eye8 = jnp.eye(8, dtype=jnp.float32)
  bdwl2 = jnp.kron(eye8, W_l2)
  bdwr2 = jnp.kron(eye8, W_r2)
  b1t = jnp.tile(b_l1, 8).reshape(1, 128)
  b2t = jnp.tile(b_l2, 8).reshape(1, 8 * _C)

  y1v, xrv = _mm1(x3, W_l1, W_r1)
  aggp, degp = _sc_l1(y1v.reshape(_NP, _H), edge3, pad64, zeros, ones)
  aggv = aggp.reshape(_NC, _NV, 128)
  degv = degp.reshape(_NC, _NV, 128)
  hv = _tc2(aggv, degv, xrv, b1t)
  agg2p = _sc_l2(hv.reshape(_NP, _H), edge3, pad64, zeros)
  gsum = jnp.kron(eye8, jnp.ones((_C, _C), jnp.float32))
  outv = _tc3(agg2p.reshape(_NC, _NV, 128), degv, hv, bdwl2, bdwr2, b2t,
              gsum)
  return outv[:_N // 8].reshape(_N, _C)
